# baseline (speedup 1.0000x reference)
import functools

import jax
import jax.numpy as jnp
from jax.experimental import pallas as pl
from jax.experimental.pallas import tpu as pltpu


def _assign_kernel(x_ref, c_ref, out_ref):
    x = x_ref[...]
    c = c_ref[...]
    nb = x.shape[0]
    k = c.shape[0]
    half = k // 2
    x2 = jnp.sum(x * x, axis=1, keepdims=True)
    c2 = jnp.sum(c * c, axis=1)[None, :]
    cneg = c * -2.0
    xcn = jax.lax.dot_general(
        x, cneg, (((1,), (1,)), ((), ())),
        preferred_element_type=jnp.float32)
    u = (x2 + c2) + xcn
    u0 = u[:, :half].T
    u1 = u[:, half:].T
    m = jnp.maximum(jnp.min(jnp.minimum(u0, u1), axis=0, keepdims=True), 0.0)
    iota = jax.lax.broadcasted_iota(
        jnp.int32, (half, 1), 0).astype(jnp.float32)
    cand = jnp.where(u0 <= m, iota,
                     jnp.where(u1 <= m, iota + float(half), float(2 * k)))
    idx = jnp.min(cand, axis=0)
    out_ref[...] = idx.astype(jnp.int32)


def kernel(X, centroids):
    n, d = X.shape
    k = centroids.shape[0]
    nb = 8192
    grid = (n // nb,)
    out = pl.pallas_call(
        _assign_kernel,
        grid=grid,
        in_specs=[
            pl.BlockSpec((nb, d), lambda i: (i, 0)),
            pl.BlockSpec((k, d), lambda i: (0, 0)),
        ],
        out_specs=pl.BlockSpec((nb,), lambda i: (i,)),
        out_shape=jax.ShapeDtypeStruct((n,), jnp.int32),
        compiler_params=pltpu.CompilerParams(
            dimension_semantics=("parallel",),
        ),
    )(X, centroids)
    return out


# speedup vs baseline: 1.0553x; 1.0553x over previous
import functools

import jax
import jax.numpy as jnp
from jax.experimental import pallas as pl
from jax.experimental.pallas import tpu as pltpu


def _assign_kernel(x_ref, c_ref, out_ref, cneg_ref, c2_ref):
    x = x_ref[...]
    nb = x.shape[0]
    k = c_ref.shape[0]
    half = k // 2

    @pl.when(pl.program_id(0) == 0)
    def _():
        c = c_ref[...]
        cneg_ref[...] = c * -2.0
        c2_ref[...] = jnp.sum(c * c, axis=1)[None, :]

    x2 = jnp.sum(x * x, axis=1, keepdims=True)
    c2 = c2_ref[...]
    cneg = cneg_ref[...]
    xcn = jax.lax.dot_general(
        x, cneg, (((1,), (1,)), ((), ())),
        preferred_element_type=jnp.float32)
    u = (x2 + c2) + xcn
    u0 = u[:, :half].T
    u1 = u[:, half:].T
    m = jnp.maximum(jnp.min(jnp.minimum(u0, u1), axis=0, keepdims=True), 0.0)
    iota = jax.lax.broadcasted_iota(
        jnp.int32, (half, 1), 0).astype(jnp.float32)
    cand = jnp.where(u0 <= m, iota,
                     jnp.where(u1 <= m, iota + float(half), float(2 * k)))
    idx = jnp.min(cand, axis=0)
    out_ref[...] = idx.astype(jnp.int32)


def kernel(X, centroids):
    n, d = X.shape
    k = centroids.shape[0]
    nb = 4096
    grid = (n // nb,)
    out = pl.pallas_call(
        _assign_kernel,
        grid=grid,
        in_specs=[
            pl.BlockSpec((nb, d), lambda i: (i, 0)),
            pl.BlockSpec((k, d), lambda i: (0, 0)),
        ],
        out_specs=pl.BlockSpec((nb,), lambda i: (i,)),
        out_shape=jax.ShapeDtypeStruct((n,), jnp.int32),
        scratch_shapes=[
            pltpu.VMEM((k, d), jnp.float32),
            pltpu.VMEM((1, k), jnp.float32),
        ],
        compiler_params=pltpu.CompilerParams(
            dimension_semantics=("arbitrary",),
        ),
    )(X, centroids)
    return out
